# baseline (speedup 1.0000x reference)
import functools

import jax
import jax.numpy as jnp
from jax import lax
from jax.experimental import pallas as pl
from jax.experimental.pallas import tpu as pltpu
from jax.experimental.pallas import tpu_sc as plsc


def _gemm_body(BR, lo_ref, e_ref, act_ref, hi_ref, x_ref, p_ref, wg_ref,
               wu_ref, wd_ref, o_ref):
    fb = pl.program_id(0)
    g = pl.program_id(1)

    @pl.when((fb == 0) & (g == 0))
    def _init():
        o_ref[...] = jnp.zeros_like(o_ref)

    @pl.when(act_ref[g] == 1)
    def _compute():
        lo = pl.multiple_of(lo_ref[g], 8)
        x = x_ref[pl.ds(lo, BR), :]
        gate = lax.dot_general(x, wg_ref[0], (((1,), (1,)), ((), ())),
                               preferred_element_type=jnp.float32)
        up = lax.dot_general(x, wu_ref[0], (((1,), (1,)), ((), ())),
                             preferred_element_type=jnp.float32)
        h = (gate * jax.nn.sigmoid(gate)) * up
        rows = lo + lax.broadcasted_iota(jnp.int32, (BR, 1), 0)
        h = h * jnp.where(rows < hi_ref[g], p_ref[pl.ds(lo, BR), :], 0.0)
        out = lax.dot_general(h, wd_ref[0], (((1,), (1,)), ((), ())),
                              preferred_element_type=jnp.float32)
        o_ref[pl.ds(lo, BR), :] += out


def kernel(hidden_states, router_weights, ori_shape, selected_experts,
           topk_map, token_per_expert, gate_weight, up_weight, down_weight):
    S, D = hidden_states.shape
    K = router_weights.shape[1]
    E = token_per_expert.shape[0]
    F = gate_weight.shape[0] // E
    KS = K * S

    BR = 512
    BF = 512
    PAD = 8
    KSP = KS + E * (PAD - 1) + BR
    KSP = ((KSP + 31) // 32) * 32
    G_MAX = KS // BR + E

    flat_e = selected_experts.T.reshape(-1).astype(jnp.int32)
    oh = (flat_e[:, None] == jnp.arange(E, dtype=jnp.int32)[None, :])
    within = jnp.sum(jnp.cumsum(oh.astype(jnp.int32), axis=0) * oh,
                     axis=1) - 1
    tpe = token_per_expert.astype(jnp.int32)
    tpe_pad = ((tpe + PAD - 1) // PAD) * PAD
    offp = jnp.concatenate([jnp.zeros((1,), jnp.int32),
                            jnp.cumsum(tpe_pad)]).astype(jnp.int32)
    rankp = offp[flat_e] + within

    flat_tok = jnp.tile(jnp.arange(S, dtype=jnp.int32), K)
    src_tok = jnp.zeros((KSP,), jnp.int32).at[rankp].set(flat_tok)
    probs_p = jnp.zeros((KSP,), jnp.float32).at[rankp].set(
        router_weights.T.reshape(-1).astype(jnp.float32))
    pos0 = rankp[:S]
    pos1 = rankp[S:]

    nblk = (tpe + BR - 1) // BR
    blk_cum = jnp.cumsum(nblk)
    nblocks = blk_cum[-1]
    gids = jnp.arange(G_MAX, dtype=jnp.int32)
    e_of_g = jnp.minimum(
        jnp.searchsorted(blk_cum, gids, side="right"), E - 1).astype(jnp.int32)
    blk_base = blk_cum - nblk
    j_of_g = gids - blk_base[e_of_g]
    lo_g = offp[e_of_g] + j_of_g * BR
    act_g = (gids < nblocks).astype(jnp.int32)
    lo_g = jnp.where(act_g == 1, lo_g, 0).astype(jnp.int32)
    hi_g = jnp.minimum(offp[e_of_g] + tpe[e_of_g], lo_g + BR)
    hi_g = jnp.where(act_g == 1, hi_g, 0).astype(jnp.int32)

    Wg3 = gate_weight.reshape(E, F, D)
    Wu3 = up_weight.reshape(E, F, D)
    Wd3 = down_weight.reshape(E, D, F)

    mesh = plsc.VectorSubcoreMesh(core_axis_name="c", subcore_axis_name="s",
                                  num_cores=2, num_subcores=16)
    NW = 32
    CH = 32
    NCHUNK = KSP // CH

    @functools.partial(
        pl.kernel,
        out_type=jax.ShapeDtypeStruct((KSP, D), jnp.float32),
        mesh=mesh,
        scratch_types=[
            pltpu.VMEM((CH,), jnp.int32),
            pltpu.VMEM((CH, D), jnp.float32),
            pltpu.SemaphoreType.DMA,
        ],
    )
    def _dispatch(x_hbm, idx_hbm, out_hbm, idx_v, rows_v, sem):
        wid = lax.axis_index("s") * 2 + lax.axis_index("c")

        def body(k, _):
            c = wid + k * NW

            @pl.when(c < NCHUNK)
            def _():
                base = c * CH
                pltpu.sync_copy(idx_hbm.at[pl.ds(base, CH)], idx_v)
                pltpu.async_copy(x_hbm.at[idx_v], rows_v, sem).wait()
                pltpu.sync_copy(rows_v, out_hbm.at[pl.ds(base, CH)])
            return 0

        lax.fori_loop(0, (NCHUNK + NW - 1) // NW, body, 0)

    grouped_x = _dispatch(hidden_states, src_tok)

    gemm_spec = pltpu.PrefetchScalarGridSpec(
        num_scalar_prefetch=4,
        grid=(F // BF, G_MAX),
        in_specs=[
            pl.BlockSpec((KSP, D), lambda fb, g, lo, e, a, hi: (0, 0)),
            pl.BlockSpec((KSP, 1), lambda fb, g, lo, e, a, hi: (0, 0)),
            pl.BlockSpec((1, BF, D), lambda fb, g, lo, e, a, hi: (e[g], fb, 0)),
            pl.BlockSpec((1, BF, D), lambda fb, g, lo, e, a, hi: (e[g], fb, 0)),
            pl.BlockSpec((1, D, BF), lambda fb, g, lo, e, a, hi: (e[g], 0, fb)),
        ],
        out_specs=pl.BlockSpec((KSP, D), lambda fb, g, lo, e, a, hi: (0, 0)),
    )
    down_out = pl.pallas_call(
        functools.partial(_gemm_body, BR),
        grid_spec=gemm_spec,
        out_shape=jax.ShapeDtypeStruct((KSP, D), jnp.float32),
        compiler_params=pltpu.CompilerParams(
            dimension_semantics=("arbitrary", "arbitrary")),
    )(lo_g, e_of_g, act_g, hi_g, grouped_x, probs_p[:, None], Wg3, Wu3, Wd3)

    CH2 = 16
    NPW = S // CH2 // NW
    idx_comb = jnp.concatenate(
        [pos0.reshape(S // CH2, CH2), pos1.reshape(S // CH2, CH2)],
        axis=1).reshape(-1)

    @functools.partial(
        pl.kernel,
        out_type=jax.ShapeDtypeStruct((S, D), jnp.float32),
        mesh=mesh,
        scratch_types=[
            pltpu.VMEM((2, 2 * CH2), jnp.int32),
            pltpu.VMEM((2, 2 * CH2, D), jnp.float32),
            pltpu.VMEM((2, CH2, D), jnp.float32),
            pltpu.SemaphoreType.DMA,
            pltpu.SemaphoreType.DMA,
            pltpu.SemaphoreType.DMA,
            pltpu.SemaphoreType.DMA,
        ],
    )
    def _combine(d_hbm, idx_hbm, out_hbm, idx_v, buf_v, out_v,
                 sg0, sg1, ss0, ss1):
        wid = lax.axis_index("s") * 2 + lax.axis_index("c")
        base_c = wid * NPW
        sgs = (sg0, sg1)
        sss = (ss0, ss1)
        store_h = [None, None]
        for k in range(NPW):
            b = k % 2
            c = base_c + k
            pltpu.sync_copy(idx_hbm.at[pl.ds(c * 2 * CH2, 2 * CH2)],
                            idx_v.at[b])
            pltpu.async_copy(d_hbm.at[idx_v.at[b]], buf_v.at[b],
                             sgs[b]).wait()
            if k >= 2:
                store_h[b].wait()

            @functools.partial(plsc.parallel_loop, 0, CH2 * (D // 16),
                               unroll=8)
            def _add(j, b=b):
                r = j // (D // 16)
                col = (j % (D // 16)) * 16
                out_v[b, r, pl.ds(col, 16)] = (
                    buf_v[b, r, pl.ds(col, 16)] +
                    buf_v[b, r + CH2, pl.ds(col, 16)])

            store_h[b] = pltpu.async_copy(
                out_v.at[b], out_hbm.at[pl.ds(c * CH2, CH2)], sss[b])
        store_h[0].wait()
        store_h[1].wait()

    final = _combine(down_out, idx_comb)
    return final + (ori_shape[0] * 0).astype(final.dtype)


# speedup vs baseline: 1.0145x; 1.0145x over previous
import functools

import jax
import jax.numpy as jnp
from jax import lax
from jax.experimental import pallas as pl
from jax.experimental.pallas import tpu as pltpu
from jax.experimental.pallas import tpu_sc as plsc


def _gemm_body(BR, lo_ref, e_ref, act_ref, hi_ref, x_ref, p_ref, wg_ref,
               wu_ref, wd_ref, o_ref):
    fb = pl.program_id(0)
    g = pl.program_id(1)

    @pl.when((fb == 0) & (g == 0))
    def _init():
        o_ref[...] = jnp.zeros_like(o_ref)

    @pl.when(act_ref[g] == 1)
    def _compute():
        lo = pl.multiple_of(lo_ref[g], 8)
        x = x_ref[pl.ds(lo, BR), :]
        gate = lax.dot_general(x, wg_ref[0], (((1,), (1,)), ((), ())),
                               preferred_element_type=jnp.float32)
        up = lax.dot_general(x, wu_ref[0], (((1,), (1,)), ((), ())),
                             preferred_element_type=jnp.float32)
        h = (gate * jax.nn.sigmoid(gate)) * up
        rows = lo + lax.broadcasted_iota(jnp.int32, (BR, 1), 0)
        h = h * jnp.where(rows < hi_ref[g], p_ref[pl.ds(lo, BR), :], 0.0)
        out = lax.dot_general(h, wd_ref[0], (((1,), (1,)), ((), ())),
                              preferred_element_type=jnp.float32)
        o_ref[pl.ds(lo, BR), :] += out


def kernel(hidden_states, router_weights, ori_shape, selected_experts,
           topk_map, token_per_expert, gate_weight, up_weight, down_weight):
    S, D = hidden_states.shape
    K = router_weights.shape[1]
    E = token_per_expert.shape[0]
    F = gate_weight.shape[0] // E
    KS = K * S

    BR = 512
    BF = 512
    PAD = 8
    KSP = KS + E * (PAD - 1) + BR
    KSP = ((KSP + 31) // 32) * 32
    G_MAX = KS // BR + E

    flat_e = selected_experts.T.reshape(-1).astype(jnp.int32)
    oh = (flat_e[:, None] == jnp.arange(E, dtype=jnp.int32)[None, :])
    CHK = 128
    NCK = KS // CHK
    oh3 = oh.astype(jnp.float32).reshape(NCK, CHK, E)
    lt = (jnp.arange(CHK)[:, None] >= jnp.arange(CHK)[None, :]).astype(
        jnp.float32)
    intra = jnp.einsum("ij,cje->cie", lt, oh3,
                       precision=lax.Precision.HIGHEST)
    lts = (jnp.arange(NCK)[:, None] > jnp.arange(NCK)[None, :]).astype(
        jnp.float32)
    base = jnp.einsum("cd,de->ce", lts, oh3.sum(axis=1),
                      precision=lax.Precision.HIGHEST)
    cum = (intra + base[:, None, :]).reshape(KS, E)
    within = jnp.round(jnp.sum(cum * oh.astype(jnp.float32), axis=1)
                       ).astype(jnp.int32) - 1
    tpe = token_per_expert.astype(jnp.int32)
    tpe_pad = ((tpe + PAD - 1) // PAD) * PAD
    offp = jnp.concatenate([jnp.zeros((1,), jnp.int32),
                            jnp.cumsum(tpe_pad)]).astype(jnp.int32)
    rankp = offp[flat_e] + within

    flat_tok = jnp.tile(jnp.arange(S, dtype=jnp.int32), K)
    src_tok = jnp.zeros((KSP,), jnp.int32).at[rankp].set(flat_tok)
    probs_p = jnp.zeros((KSP,), jnp.float32).at[rankp].set(
        router_weights.T.reshape(-1).astype(jnp.float32))
    pos0 = rankp[:S]
    pos1 = rankp[S:]

    nblk = (tpe + BR - 1) // BR
    blk_cum = jnp.cumsum(nblk)
    nblocks = blk_cum[-1]
    gids = jnp.arange(G_MAX, dtype=jnp.int32)
    e_of_g = jnp.minimum(
        jnp.searchsorted(blk_cum, gids, side="right"), E - 1).astype(jnp.int32)
    blk_base = blk_cum - nblk
    j_of_g = gids - blk_base[e_of_g]
    lo_g = offp[e_of_g] + j_of_g * BR
    act_g = (gids < nblocks).astype(jnp.int32)
    lo_g = jnp.where(act_g == 1, lo_g, 0).astype(jnp.int32)
    hi_g = jnp.minimum(offp[e_of_g] + tpe[e_of_g], lo_g + BR)
    hi_g = jnp.where(act_g == 1, hi_g, 0).astype(jnp.int32)

    Wg3 = gate_weight.reshape(E, F, D)
    Wu3 = up_weight.reshape(E, F, D)
    Wd3 = down_weight.reshape(E, D, F)

    mesh = plsc.VectorSubcoreMesh(core_axis_name="c", subcore_axis_name="s",
                                  num_cores=2, num_subcores=16)
    NW = 32
    CH = 32
    NCHUNK = KSP // CH

    @functools.partial(
        pl.kernel,
        out_type=jax.ShapeDtypeStruct((KSP, D), jnp.float32),
        mesh=mesh,
        scratch_types=[
            pltpu.VMEM((CH,), jnp.int32),
            pltpu.VMEM((CH, D), jnp.float32),
            pltpu.SemaphoreType.DMA,
        ],
    )
    def _dispatch(x_hbm, idx_hbm, out_hbm, idx_v, rows_v, sem):
        wid = lax.axis_index("s") * 2 + lax.axis_index("c")

        def body(k, _):
            c = wid + k * NW

            @pl.when(c < NCHUNK)
            def _():
                base = c * CH
                pltpu.sync_copy(idx_hbm.at[pl.ds(base, CH)], idx_v)
                pltpu.async_copy(x_hbm.at[idx_v], rows_v, sem).wait()
                pltpu.sync_copy(rows_v, out_hbm.at[pl.ds(base, CH)])
            return 0

        lax.fori_loop(0, (NCHUNK + NW - 1) // NW, body, 0)

    grouped_x = _dispatch(hidden_states, src_tok)

    gemm_spec = pltpu.PrefetchScalarGridSpec(
        num_scalar_prefetch=4,
        grid=(F // BF, G_MAX),
        in_specs=[
            pl.BlockSpec((KSP, D), lambda fb, g, lo, e, a, hi: (0, 0)),
            pl.BlockSpec((KSP, 1), lambda fb, g, lo, e, a, hi: (0, 0)),
            pl.BlockSpec((1, BF, D), lambda fb, g, lo, e, a, hi: (e[g], fb, 0)),
            pl.BlockSpec((1, BF, D), lambda fb, g, lo, e, a, hi: (e[g], fb, 0)),
            pl.BlockSpec((1, D, BF), lambda fb, g, lo, e, a, hi: (e[g], 0, fb)),
        ],
        out_specs=pl.BlockSpec((KSP, D), lambda fb, g, lo, e, a, hi: (0, 0)),
    )
    down_out = pl.pallas_call(
        functools.partial(_gemm_body, BR),
        grid_spec=gemm_spec,
        out_shape=jax.ShapeDtypeStruct((KSP, D), jnp.float32),
        compiler_params=pltpu.CompilerParams(
            dimension_semantics=("arbitrary", "arbitrary")),
    )(lo_g, e_of_g, act_g, hi_g, grouped_x, probs_p[:, None], Wg3, Wu3, Wd3)

    CH2 = 16
    NPW = S // CH2 // NW
    idx_comb = jnp.concatenate(
        [pos0.reshape(S // CH2, CH2), pos1.reshape(S // CH2, CH2)],
        axis=1).reshape(-1)

    @functools.partial(
        pl.kernel,
        out_type=jax.ShapeDtypeStruct((S, D), jnp.float32),
        mesh=mesh,
        scratch_types=[
            pltpu.VMEM((2, 2 * CH2), jnp.int32),
            pltpu.VMEM((2, 2 * CH2, D), jnp.float32),
            pltpu.VMEM((2, CH2, D), jnp.float32),
            pltpu.SemaphoreType.DMA,
            pltpu.SemaphoreType.DMA,
            pltpu.SemaphoreType.DMA,
            pltpu.SemaphoreType.DMA,
        ],
    )
    def _combine(d_hbm, idx_hbm, out_hbm, idx_v, buf_v, out_v,
                 sg0, sg1, ss0, ss1):
        wid = lax.axis_index("s") * 2 + lax.axis_index("c")
        base_c = wid * NPW
        sgs = (sg0, sg1)
        sss = (ss0, ss1)
        store_h = [None, None]
        for k in range(NPW):
            b = k % 2
            c = base_c + k
            pltpu.sync_copy(idx_hbm.at[pl.ds(c * 2 * CH2, 2 * CH2)],
                            idx_v.at[b])
            pltpu.async_copy(d_hbm.at[idx_v.at[b]], buf_v.at[b],
                             sgs[b]).wait()
            if k >= 2:
                store_h[b].wait()

            @functools.partial(plsc.parallel_loop, 0, CH2 * (D // 16),
                               unroll=8)
            def _add(j, b=b):
                r = j // (D // 16)
                col = (j % (D // 16)) * 16
                out_v[b, r, pl.ds(col, 16)] = (
                    buf_v[b, r, pl.ds(col, 16)] +
                    buf_v[b, r + CH2, pl.ds(col, 16)])

            store_h[b] = pltpu.async_copy(
                out_v.at[b], out_hbm.at[pl.ds(c * CH2, CH2)], sss[b])
        store_h[0].wait()
        store_h[1].wait()

    final = _combine(down_out, idx_comb)
    return final + (ori_shape[0] * 0).astype(final.dtype)
